# baseline (speedup 1.0000x reference)
import functools

import jax
import jax.numpy as jnp
from jax import lax
from jax.experimental import pallas as pl
from jax.experimental.pallas import tpu as pltpu
from jax.experimental.pallas import tpu_sc as plsc

_INFO = plsc.get_sparse_core_info()
_NC = _INFO.num_cores
_NS = _INFO.num_subcores
_NW = _NC * _NS
_L = 16


def _emb_gather_t(wpad, x_t, s_len, d, b):
    bw = b // _NW
    assert s_len % 2 == 0

    mesh = plsc.VectorSubcoreMesh(core_axis_name="c", subcore_axis_name="s")

    @functools.partial(
        pl.kernel,
        mesh=mesh,
        out_type=jax.ShapeDtypeStruct((s_len, d, b), jnp.float32),
        scratch_types=(
            [pltpu.VMEM((s_len, bw), jnp.int32),
             pltpu.VMEM((2, bw, 128), jnp.float32),
             pltpu.VMEM((2, d, bw), jnp.float32)]
            + [pltpu.SemaphoreType.DMA] * 4
        ),
        compiler_params=pltpu.CompilerParams(needs_layout_passes=False),
    )
    def k(wpad_hbm, xt_hbm, out_hbm, idx_v, r_v, m_v, g0, g1, t0, t1):
        gsem = (g0, g1)
        ssem = (t0, t1)
        wid = lax.axis_index("s") * _NC + lax.axis_index("c")
        b0 = wid * bw
        pltpu.sync_copy(xt_hbm.at[:, pl.ds(b0, bw)], idx_v)

        def start_gather(buf, s):
            pltpu.async_copy(wpad_hbm.at[idx_v.at[s]], r_v.at[buf], gsem[buf])

        def wait_gather(buf):
            pltpu.make_async_copy(
                wpad_hbm.at[pl.ds(0, bw)], r_v.at[buf], gsem[buf]
            ).wait()

        def start_store(buf, s):
            pltpu.async_copy(
                m_v.at[buf], out_hbm.at[s, :, pl.ds(b0, bw)], ssem[buf]
            )

        def wait_store(buf):
            pltpu.make_async_copy(
                m_v.at[buf], out_hbm.at[0, :, pl.ds(0, bw)], ssem[buf]
            ).wait()

        rows = [
            jnp.full((_L,), 16 * j, jnp.int32) + lax.iota(jnp.int32, _L)
            for j in range(bw // _L)
        ]

        start_gather(0, 0)
        start_gather(1, 1)

        def outer(i, carry):
            for buf in range(2):
                s = 2 * i + buf
                wait_gather(buf)

                @pl.when(i >= 1)
                def _():
                    wait_store(buf)

                for dd in range(d):
                    cols = jnp.full((_L,), dd, jnp.int32)
                    for j in range(bw // _L):
                        vals = plsc.load_gather(r_v.at[buf], [rows[j], cols])
                        m_v[buf, dd, pl.ds(16 * j, _L)] = vals
                start_store(buf, s)
                nxt = jnp.minimum(s + 2, s_len - 1)
                start_gather(buf, nxt)
            return carry

        lax.fori_loop(0, s_len // 2, outer, 0)

        for buf in range(2):
            wait_gather(buf)
            wait_store(buf)

    return k(wpad, x_t)


def kernel(x, emb_weight):
    b, s = x.shape
    v, d = emb_weight.shape
    wpad = jnp.pad(emb_weight, ((0, 0), (0, 128 - d)))
    x_t = x.T.astype(jnp.int32)
    out_t = _emb_gather_t(wpad, x_t, s, d, b)
    return out_t.transpose(2, 0, 1)


# speedup vs baseline: 1.9235x; 1.9235x over previous
import functools

import jax
import jax.numpy as jnp
from jax import lax
from jax.experimental import pallas as pl
from jax.experimental.pallas import tpu as pltpu
from jax.experimental.pallas import tpu_sc as plsc

_INFO = plsc.get_sparse_core_info()
_NC = _INFO.num_cores
_NS = _INFO.num_subcores
_NW = _NC * _NS
_L = 16


def _emb_gather_t(wpad, x_t, s_len, d, b):
    bw = b // _NW
    assert s_len % 2 == 0

    mesh = plsc.VectorSubcoreMesh(core_axis_name="c", subcore_axis_name="s")

    @functools.partial(
        pl.kernel,
        mesh=mesh,
        out_type=jax.ShapeDtypeStruct((s_len, d, b), jnp.float32),
        scratch_types=(
            [pltpu.VMEM((s_len, bw), jnp.int32),
             pltpu.VMEM((2, bw, 128), jnp.float32),
             pltpu.VMEM((2, d, bw), jnp.float32)]
            + [pltpu.SemaphoreType.DMA] * 4
        ),
        compiler_params=pltpu.CompilerParams(needs_layout_passes=False),
    )
    def k(wpad_hbm, xt_hbm, out_hbm, idx_v, r_v, m_v, g0, g1, t0, t1):
        gsem = (g0, g1)
        ssem = (t0, t1)
        wid = lax.axis_index("s") * _NC + lax.axis_index("c")
        b0 = wid * bw
        pltpu.sync_copy(xt_hbm.at[:, pl.ds(b0, bw)], idx_v)

        def start_gather(buf, s):
            pltpu.async_copy(wpad_hbm.at[idx_v.at[s]], r_v.at[buf], gsem[buf])

        def wait_gather(buf):
            pltpu.make_async_copy(
                wpad_hbm.at[pl.ds(0, bw)], r_v.at[buf], gsem[buf]
            ).wait()

        def start_store(buf, s):
            pltpu.async_copy(
                m_v.at[buf], out_hbm.at[s, :, pl.ds(b0, bw)], ssem[buf]
            )

        def wait_store(buf):
            pltpu.make_async_copy(
                m_v.at[buf], out_hbm.at[0, :, pl.ds(0, bw)], ssem[buf]
            ).wait()

        lane = lax.iota(jnp.int32, _L)
        perm = [(lane + off) % _L for off in range(_L)]

        start_gather(0, 0)
        start_gather(1, 1)

        def outer(i, carry):
            for buf in range(2):
                s = 2 * i + buf
                wait_gather(buf)

                @pl.when(i >= 1)
                def _():
                    wait_store(buf)

                r2 = r_v.at[buf]
                m2 = m_v.at[buf]

                def tr_body(j, tcarry):
                    rowsj = lane + 16 * j
                    for dd0 in range(0, d, _L):
                        for off in range(_L):
                            cols = perm[off] + dd0
                            vals = plsc.load_gather(r2, [rowsj, cols])
                            plsc.store_scatter(m2, [cols, rowsj], vals)
                    return tcarry

                lax.fori_loop(0, bw // _L, tr_body, 0)
                start_store(buf, s)
                nxt = jnp.minimum(s + 2, s_len - 1)
                start_gather(buf, nxt)
            return carry

        lax.fori_loop(0, s_len // 2, outer, 0)

        for buf in range(2):
            wait_gather(buf)
            wait_store(buf)

    return k(wpad, x_t)


def kernel(x, emb_weight):
    b, s = x.shape
    v, d = emb_weight.shape
    wpad = jnp.pad(emb_weight, ((0, 0), (0, 128 - d)))
    x_t = x.T.astype(jnp.int32)
    out_t = _emb_gather_t(wpad, x_t, s, d, b)
    return out_t.transpose(2, 0, 1)


# speedup vs baseline: 2.0378x; 1.0594x over previous
import functools

import jax
import jax.numpy as jnp
from jax import lax
from jax.experimental import pallas as pl
from jax.experimental.pallas import tpu as pltpu
from jax.experimental.pallas import tpu_sc as plsc

_INFO = plsc.get_sparse_core_info()
_NC = _INFO.num_cores
_NS = _INFO.num_subcores
_NW = _NC * _NS
_L = 16


def _emb_gather_t(wpad, x_t, s_len, d, b):
    bw = b // _NW
    assert s_len % 2 == 0

    mesh = plsc.VectorSubcoreMesh(core_axis_name="c", subcore_axis_name="s")

    @functools.partial(
        pl.kernel,
        mesh=mesh,
        out_type=jax.ShapeDtypeStruct((s_len, d, b), jnp.float32),
        scratch_types=(
            [pltpu.VMEM((s_len, bw), jnp.int32),
             pltpu.VMEM((2, bw, 128), jnp.float32),
             pltpu.VMEM((2, d, bw), jnp.float32)]
            + [pltpu.SemaphoreType.DMA] * 4
        ),
        compiler_params=pltpu.CompilerParams(needs_layout_passes=False),
    )
    def k(wpad_hbm, xt_hbm, out_hbm, idx_v, r_v, m_v, g0, g1, t0, t1):
        gsem = (g0, g1)
        ssem = (t0, t1)
        wid = lax.axis_index("s") * _NC + lax.axis_index("c")
        b0 = wid * bw
        pltpu.sync_copy(xt_hbm.at[:, pl.ds(b0, bw)], idx_v)

        def start_gather(buf, s):
            pltpu.async_copy(wpad_hbm.at[idx_v.at[s]], r_v.at[buf], gsem[buf])

        def wait_gather(buf):
            pltpu.make_async_copy(
                wpad_hbm.at[pl.ds(0, bw)], r_v.at[buf], gsem[buf]
            ).wait()

        def start_store(buf, s):
            pltpu.async_copy(
                m_v.at[buf], out_hbm.at[s, :, pl.ds(b0, bw)], ssem[buf]
            )

        def wait_store(buf):
            pltpu.make_async_copy(
                m_v.at[buf], out_hbm.at[0, :, pl.ds(0, bw)], ssem[buf]
            ).wait()

        lane = lax.iota(jnp.int32, _L)
        perm = [(lane + off) % _L for off in range(_L)]

        start_gather(0, 0)
        start_gather(1, 1)

        def outer(i, carry):
            for buf in range(2):
                s = 2 * i + buf
                wait_gather(buf)

                @pl.when(i >= 1)
                def _():
                    wait_store(buf)

                r2 = r_v.at[buf]
                m2 = m_v.at[buf]

                @plsc.parallel_loop(0, bw // _L, unroll=2)
                def _(j):
                    rowsj = lane + 16 * j
                    for dd0 in range(0, d, _L):
                        for off in range(_L):
                            cols = perm[off] + dd0
                            vals = plsc.load_gather(r2, [rowsj, cols])
                            plsc.store_scatter(m2, [cols, rowsj], vals)
                start_store(buf, s)
                nxt = jnp.minimum(s + 2, s_len - 1)
                start_gather(buf, nxt)
            return carry

        lax.fori_loop(0, s_len // 2, outer, 0)

        for buf in range(2):
            wait_gather(buf)
            wait_store(buf)

    return k(wpad, x_t)


def kernel(x, emb_weight):
    b, s = x.shape
    v, d = emb_weight.shape
    wpad = jnp.pad(emb_weight, ((0, 0), (0, 128 - d)))
    x_t = x.T.astype(jnp.int32)
    out_t = _emb_gather_t(wpad, x_t, s, d, b)
    return out_t.transpose(2, 0, 1)


# speedup vs baseline: 3.0528x; 1.4981x over previous
import functools

import jax
import jax.numpy as jnp
from jax import lax
from jax.experimental import pallas as pl
from jax.experimental.pallas import tpu as pltpu
from jax.experimental.pallas import tpu_sc as plsc

_INFO = plsc.get_sparse_core_info()
_NC = _INFO.num_cores
_NS = _INFO.num_subcores
_NW = _NC * _NS
_L = 16


def _emb_gather_t(wpad, x_t, s_len, d, b):
    bw = b // _NW
    assert s_len % 2 == 0

    mesh = plsc.VectorSubcoreMesh(core_axis_name="c", subcore_axis_name="s")

    @functools.partial(
        pl.kernel,
        mesh=mesh,
        out_type=jax.ShapeDtypeStruct((s_len, d, b), jnp.float32),
        scratch_types=(
            [pltpu.VMEM((s_len, bw), jnp.int32),
             pltpu.VMEM((2, bw, 128), jnp.float32),
             pltpu.VMEM((2, d, bw), jnp.float32)]
            + [pltpu.SemaphoreType.DMA] * 4
        ),
        compiler_params=pltpu.CompilerParams(needs_layout_passes=False),
    )
    def k(wpad_hbm, xt_hbm, out_hbm, idx_v, r_v, m_v, g0, g1, t0, t1):
        gsem = (g0, g1)
        ssem = (t0, t1)
        wid = lax.axis_index("s") * _NC + lax.axis_index("c")
        b0 = wid * bw
        pltpu.sync_copy(xt_hbm.at[:, pl.ds(b0, bw)], idx_v)

        def start_gather(buf, s):
            pltpu.async_copy(wpad_hbm.at[idx_v.at[s]], r_v.at[buf], gsem[buf])

        def wait_gather(buf):
            pltpu.make_async_copy(
                wpad_hbm.at[pl.ds(0, bw)], r_v.at[buf], gsem[buf]
            ).wait()

        def start_store(buf, s):
            pltpu.async_copy(
                m_v.at[buf], out_hbm.at[s, :, pl.ds(b0, bw)], ssem[buf]
            )

        def wait_store(buf):
            pltpu.make_async_copy(
                m_v.at[buf], out_hbm.at[0, :, pl.ds(0, bw)], ssem[buf]
            ).wait()

        lane = lax.iota(jnp.int32, _L)
        perm = [(lane + off) % _L for off in range(_L)]

        start_gather(0, 0)
        start_gather(1, 1)

        def outer(i, carry):
            for buf in range(2):
                s = 2 * i + buf
                wait_gather(buf)

                @pl.when(i >= 1)
                def _():
                    wait_store(buf)

                r2 = r_v.at[buf]
                m2 = m_v.at[buf]

                @plsc.parallel_loop(0, bw // _L, unroll=4)
                def _(j):
                    rowsj = lane + 16 * j
                    for dd0 in range(0, d, _L):
                        for off in range(_L):
                            cols = perm[off] + dd0
                            vals = plsc.load_gather(r2, [rowsj, cols])
                            plsc.store_scatter(m2, [cols, rowsj], vals)
                start_store(buf, s)
                nxt = jnp.minimum(s + 2, s_len - 1)
                start_gather(buf, nxt)
            return carry

        lax.fori_loop(0, s_len // 2, outer, 0)

        for buf in range(2):
            wait_gather(buf)
            wait_store(buf)

    return k(wpad, x_t)


def kernel(x, emb_weight):
    b, s = x.shape
    v, d = emb_weight.shape
    wpad = jnp.pad(emb_weight, ((0, 0), (0, 128 - d)))
    x_t = x.T.astype(jnp.int32)
    out_t = _emb_gather_t(wpad, x_t, s, d, b)
    return out_t.transpose(2, 0, 1)


# speedup vs baseline: 3.2001x; 1.0483x over previous
import functools

import jax
import jax.numpy as jnp
from jax import lax
from jax.experimental import pallas as pl
from jax.experimental.pallas import tpu as pltpu
from jax.experimental.pallas import tpu_sc as plsc

_INFO = plsc.get_sparse_core_info()
_NC = _INFO.num_cores
_NS = _INFO.num_subcores
_NW = _NC * _NS
_L = 16


def _emb_gather_t(wpad, x_t, s_len, d, b):
    bw = b // _NW
    nbuf = 3
    nit = (s_len + nbuf - 1) // nbuf

    mesh = plsc.VectorSubcoreMesh(core_axis_name="c", subcore_axis_name="s")

    @functools.partial(
        pl.kernel,
        mesh=mesh,
        out_type=jax.ShapeDtypeStruct((s_len, d, b), jnp.float32),
        scratch_types=(
            [pltpu.VMEM((s_len, bw), jnp.int32),
             pltpu.VMEM((nbuf, bw, 128), jnp.float32),
             pltpu.VMEM((nbuf, d, bw), jnp.float32)]
            + [pltpu.SemaphoreType.DMA] * (2 * nbuf)
        ),
        compiler_params=pltpu.CompilerParams(needs_layout_passes=False),
    )
    def k(wpad_hbm, xt_hbm, out_hbm, idx_v, r_v, m_v, *sems):
        gsem = sems[:nbuf]
        ssem = sems[nbuf:]
        wid = lax.axis_index("s") * _NC + lax.axis_index("c")
        b0 = wid * bw
        pltpu.sync_copy(xt_hbm.at[:, pl.ds(b0, bw)], idx_v)

        def start_gather(buf, s):
            pltpu.async_copy(wpad_hbm.at[idx_v.at[s]], r_v.at[buf], gsem[buf])

        def wait_gather(buf):
            pltpu.make_async_copy(
                wpad_hbm.at[pl.ds(0, bw)], r_v.at[buf], gsem[buf]
            ).wait()

        def start_store(buf, s):
            pltpu.async_copy(
                m_v.at[buf], out_hbm.at[s, :, pl.ds(b0, bw)], ssem[buf]
            )

        def wait_store(buf):
            pltpu.make_async_copy(
                m_v.at[buf], out_hbm.at[0, :, pl.ds(0, bw)], ssem[buf]
            ).wait()

        lane = lax.iota(jnp.int32, _L)
        perm = [(lane + off) % _L for off in range(_L)]

        for buf in range(nbuf):
            start_gather(buf, buf)

        def outer(i, carry):
            for buf in range(nbuf):
                s = jnp.minimum(nbuf * i + buf, s_len - 1)
                wait_gather(buf)

                @pl.when(i >= 1)
                def _():
                    wait_store(buf)

                r2 = r_v.at[buf]
                m2 = m_v.at[buf]

                @plsc.parallel_loop(0, bw // _L, unroll=4)
                def _(j):
                    rowsj = lane + 16 * j
                    for dd0 in range(0, d, _L):
                        for off in range(_L):
                            cols = perm[off] + dd0
                            vals = plsc.load_gather(r2, [rowsj, cols])
                            plsc.store_scatter(m2, [cols, rowsj], vals)
                nxt = jnp.minimum(s + nbuf, s_len - 1)
                start_gather(buf, nxt)
                start_store(buf, s)
            return carry

        lax.fori_loop(0, nit, outer, 0)

        for buf in range(nbuf):
            wait_gather(buf)
            wait_store(buf)

    return k(wpad, x_t)


def kernel(x, emb_weight):
    b, s = x.shape
    v, d = emb_weight.shape
    wpad = jnp.pad(emb_weight, ((0, 0), (0, 128 - d)))
    x_t = x.T.astype(jnp.int32)
    out_t = _emb_gather_t(wpad, x_t, s, d, b)
    return out_t.transpose(2, 0, 1)
